# baseline (speedup 1.0000x reference)
import jax
import jax.numpy as jnp
from jax.experimental import pallas as pl
from jax.experimental.pallas import tpu as pltpu

H = 16
C = 128


def _mega_kernel(cl_ref, q_ref, wq_ref, wkv_ref, wp_ref, bp_ref, out_ref):
    n, d = q_ref.shape[1], q_ref.shape[2]
    nb = cl_ref.shape[0]
    dh = d // H
    x = q_ref[0].astype(jnp.bfloat16)
    qh = jax.lax.dot_general(
        x, wq_ref[...], (((1,), (1,)), ((), ())),
        preferred_element_type=jnp.float32).astype(jnp.bfloat16)

    iota = jax.lax.broadcasted_iota(jnp.int32, (C, n), 0)
    log2e = 1.4426950408889634
    ohs = [(iota == cl_ref[r]).astype(jnp.bfloat16) for r in range(nb)]
    oh_all = jnp.concatenate(ohs, axis=0)
    cnt_col = jnp.sum(oh_all.astype(jnp.float32), axis=1, keepdims=True)
    xs = jax.lax.dot_general(
        oh_all, x, (((1,), (0,)), ((), ())),
        preferred_element_type=jnp.float32).astype(jnp.bfloat16)
    cents = jax.lax.dot_general(
        xs, wkv_ref[...], (((1,), (1,)), ((), ())),
        preferred_element_type=jnp.float32)

    w_col = jnp.where(cnt_col > 0, 1.0 / cnt_col, 0.0)
    a_col = w_col * (log2e * jax.lax.rsqrt(jnp.float32(dh)))
    lc_rows = [jnp.log2(cnt_col[r * C:(r + 1) * C]).reshape(1, C)
               for r in range(nb)]
    dcol = (jax.lax.broadcasted_iota(jnp.int32, (C, dh), 1) == 0
            ).astype(jnp.bfloat16)

    outs = []
    for h in range(H):
        r = h % nb
        rs = slice(r * C, (r + 1) * C)
        qh_h = qh[:, h * dh:(h + 1) * dh]
        kc = (cents[rs, h * dh:(h + 1) * dh]
              * a_col[rs]).astype(jnp.bfloat16)
        vc = (cents[rs, d + h * dh:d + (h + 1) * dh]
              * w_col[rs]).astype(jnp.bfloat16)
        vc_ext = jnp.concatenate([vc, dcol], axis=1)
        t = jax.lax.dot_general(
            qh_h, kc, (((1,), (1,)), ((), ())),
            preferred_element_type=jnp.float32) + lc_rows[r]
        m = jnp.max(t, axis=1, keepdims=True)
        e = jnp.exp2(t - m).astype(jnp.bfloat16)
        nd = jax.lax.dot_general(
            e, vc_ext, (((1,), (0,)), ((), ())),
            preferred_element_type=jnp.float32)
        num = nd[:, :dh]
        denom = nd[:, dh:dh + 1]
        outs.append((num * (1.0 / denom)).astype(jnp.bfloat16))
    ao = jnp.concatenate(outs, axis=1)

    out_ref[0] = jax.lax.dot_general(
        ao, wp_ref[...], (((1,), (1,)), ((), ())),
        preferred_element_type=jnp.float32) + bp_ref[...]


def kernel(cluster, q, Wq, Wkv, Wp, bp):
    B, N, D = q.shape
    cl3 = cluster.reshape(B, 1, N)
    bp2 = bp.reshape(1, D)

    out = pl.pallas_call(
        _mega_kernel,
        grid=(B,),
        in_specs=[
            pl.BlockSpec((B, 1, N), lambda b: (0, 0, 0)),
            pl.BlockSpec((1, N, D), lambda b: (b, 0, 0)),
            pl.BlockSpec((D, D), lambda b: (0, 0)),
            pl.BlockSpec((2 * D, D), lambda b: (0, 0)),
            pl.BlockSpec((D, D), lambda b: (0, 0)),
            pl.BlockSpec((1, D), lambda b: (0, 0)),
        ],
        out_specs=pl.BlockSpec((1, N, D), lambda b: (b, 0, 0)),
        out_shape=jax.ShapeDtypeStruct((B, N, D), jnp.float32),
    )(cl3, q, Wq.astype(jnp.bfloat16), Wkv.astype(jnp.bfloat16),
      Wp.astype(jnp.bfloat16), bp2)

    return out
